# baseline (speedup 1.0000x reference)
import jax
import jax.numpy as jnp
from jax.experimental import pallas as pl

TH = 0.85
BI = 512


def _qlayer_kern(x_blk_ref, x_ref, w_ref, b_ref, out_ref):
    xb = x_blk_ref[:]
    xa = x_ref[:]
    f = xa.shape[1]
    gram = jnp.dot(xb, xa.T, preferred_element_type=jnp.float32,
                   precision=jax.lax.Precision.HIGHEST)
    c = gram * gram >= TH
    rxs = jnp.sum(xa, axis=1)
    t_d = jnp.sum(jnp.where(c, rxs[None, :], 0.0), axis=1)
    cnt_d = jnp.sum(c.astype(jnp.float32), axis=1)
    sq = jnp.sum(xb * xb, axis=1)
    diag_c = (sq * sq >= TH).astype(jnp.float32)
    rxs_b = jnp.sum(xb, axis=1)
    t = t_d - diag_c * rxs_b
    cnt = cnt_d - diag_c
    scalar = jnp.where(cnt > 0.5, t / f, 0.0)
    wsum = jnp.sum(w_ref[:], axis=1)
    out_ref[:] = scalar[:, None] * wsum[None, :] + b_ref[0, :][None, :]


@jax.jit
def kernel(x, W, b):
    n, f = x.shape
    h = W.shape[0]
    b2 = b.reshape(1, h)
    return pl.pallas_call(
        _qlayer_kern,
        grid=(n // BI,),
        in_specs=[
            pl.BlockSpec((BI, f), lambda i: (i, 0)),
            pl.BlockSpec((n, f), lambda i: (0, 0)),
            pl.BlockSpec((h, f), lambda i: (0, 0)),
            pl.BlockSpec((1, h), lambda i: (0, 0)),
        ],
        out_specs=pl.BlockSpec((BI, h), lambda i: (i, 0)),
        out_shape=jax.ShapeDtypeStruct((n, h), jnp.float32),
    )(x, x, W, b2)


# speedup vs baseline: 2.6217x; 2.6217x over previous
import jax
import jax.numpy as jnp
from jax.experimental import pallas as pl

TH = 0.85
BI = 512


def _qlayer_kern(x_blk_ref, x_ref, w_ref, b_ref, out_ref):
    xb = x_blk_ref[:]
    xa = x_ref[:]
    f = xa.shape[1]
    gram = jnp.dot(xb, xa.T, preferred_element_type=jnp.float32)
    c = gram * gram >= TH
    xaq = xa.astype(jnp.bfloat16).astype(jnp.float32)
    rxs = jnp.sum(xaq, axis=1)
    t_d = jnp.sum(jnp.where(c, rxs[None, :], 0.0), axis=1)
    cnt_d = jnp.sum(c.astype(jnp.float32), axis=1)
    sq = jnp.sum(xb * xb, axis=1)
    diag_c = (sq * sq >= TH).astype(jnp.float32)
    rxs_b = jnp.sum(xb.astype(jnp.bfloat16).astype(jnp.float32), axis=1)
    t = t_d - diag_c * rxs_b
    cnt = cnt_d - diag_c
    scalar = jnp.where(cnt > 0.5, t / f, 0.0)
    scalar_q = scalar.astype(jnp.bfloat16).astype(jnp.float32)
    wsum = jnp.sum(w_ref[:].astype(jnp.bfloat16).astype(jnp.float32), axis=1)
    out_ref[:] = scalar_q[:, None] * wsum[None, :] + b_ref[0, :][None, :]


@jax.jit
def kernel(x, W, b):
    n, f = x.shape
    h = W.shape[0]
    b2 = b.reshape(1, h)
    return pl.pallas_call(
        _qlayer_kern,
        grid=(n // BI,),
        in_specs=[
            pl.BlockSpec((BI, f), lambda i: (i, 0)),
            pl.BlockSpec((n, f), lambda i: (0, 0)),
            pl.BlockSpec((h, f), lambda i: (0, 0)),
            pl.BlockSpec((1, h), lambda i: (0, 0)),
        ],
        out_specs=pl.BlockSpec((BI, h), lambda i: (i, 0)),
        out_shape=jax.ShapeDtypeStruct((n, h), jnp.float32),
    )(x, x, W, b2)


# speedup vs baseline: 3.5106x; 1.3391x over previous
import jax
import jax.numpy as jnp
from jax.experimental import pallas as pl
from jax.experimental.pallas import tpu as pltpu

TH = 0.85
BI = 512


def _qlayer_kern(x_blk_ref, x_ref, w_ref, b_ref, out_ref, rxs_ref, wsum_ref):
    i = pl.program_id(0)
    xb = x_blk_ref[:]
    f = x_ref.shape[1]

    @pl.when(i == 0)
    def _prep():
        xaq = x_ref[:].astype(jnp.bfloat16).astype(jnp.float32)
        rxs_ref[0, :] = jnp.sum(xaq, axis=1)
        wq = w_ref[:].astype(jnp.bfloat16).astype(jnp.float32)
        wsum_ref[0, :] = jnp.sum(wq, axis=1)

    gram = jnp.dot(xb, x_ref[:].T, preferred_element_type=jnp.float32)
    c = gram * gram >= TH
    rxs = rxs_ref[0, :]
    t_d = jnp.sum(jnp.where(c, rxs[None, :], 0.0), axis=1)
    sq = jnp.sum(xb * xb, axis=1)
    diag_c = (sq * sq >= TH).astype(jnp.float32)
    rxs_b = rxs_ref[0, pl.ds(i * BI, BI)]
    t = t_d - diag_c * rxs_b
    scalar = (t / f).astype(jnp.bfloat16).astype(jnp.float32)
    out_ref[:] = scalar[:, None] * wsum_ref[0, :][None, :] + b_ref[0, :][None, :]


@jax.jit
def kernel(x, W, b):
    n, f = x.shape
    h = W.shape[0]
    b2 = b.reshape(1, h)
    return pl.pallas_call(
        _qlayer_kern,
        grid=(n // BI,),
        in_specs=[
            pl.BlockSpec((BI, f), lambda i: (i, 0)),
            pl.BlockSpec((n, f), lambda i: (0, 0)),
            pl.BlockSpec((h, f), lambda i: (0, 0)),
            pl.BlockSpec((1, h), lambda i: (0, 0)),
        ],
        out_specs=pl.BlockSpec((BI, h), lambda i: (i, 0)),
        out_shape=jax.ShapeDtypeStruct((n, h), jnp.float32),
        scratch_shapes=[
            pltpu.VMEM((1, n), jnp.float32),
            pltpu.VMEM((1, h), jnp.float32),
        ],
    )(x, x, W, b2)
